# baseline (speedup 1.0000x reference)
import jax
import jax.numpy as jnp
from jax import lax
from jax.experimental import pallas as pl
from jax.experimental.pallas import tpu as pltpu
from jax.experimental.pallas import tpu_sc as plsc

_B = 16384
_D = 32
_NT = 4
_V = 100000
_NC = 2
_NS = 16
_NW = _NC * _NS
_BPW = _B // _NW
_L = 16
_C = 128
_NCH = _V // _C
_TAIL = _V - _NCH * _C


def _pack_body(t0, t1, t2, t3, big_hbm, bufs, cats, gsem, ssem):
    tables = (t0, t1, t2, t3)
    wid = lax.axis_index("s") * _NC + lax.axis_index("c")
    lane = jnp.arange(_L, dtype=jnp.int32)

    def off_of(c):
        return pl.multiple_of(c * _C, _C)

    def fire_reads(c, slot):
        off = off_of(c)
        for t in range(_NT):
            pltpu.async_copy(
                tables[t].at[:, pl.ds(off, _C)], bufs.at[t, slot], gsem
            )

    def wait_reads(c, slot):
        off = off_of(c)
        for t in range(_NT):
            pltpu.make_async_copy(
                tables[t].at[:, pl.ds(off, _C)], bufs.at[t, slot], gsem
            ).wait()

    def fire_write(c, slot):
        off = off_of(c)
        pltpu.async_copy(cats.at[slot], big_hbm.at[pl.ds(off, _C), :], ssem)

    def wait_write(c, slot):
        off = off_of(c)
        pltpu.make_async_copy(
            cats.at[slot], big_hbm.at[pl.ds(off, _C), :], ssem
        ).wait()

    def transpose_into(slot):
        for t in range(_NT):
            for d in range(_D):
                col = jnp.full((_L,), t * _D + d, dtype=jnp.int32)
                for g in range(_C // _L):
                    vals = bufs[t, slot, d, pl.ds(g * _L, _L)]
                    plsc.store_scatter(
                        cats.at[slot], [g * _L + lane, col], vals
                    )

    nmine = (_NCH - wid + _NW - 1) // _NW

    @pl.when(nmine >= 1)
    def _():
        fire_reads(wid, 0)

    def loop_body(k, _):
        c = wid + k * _NW
        slot = lax.rem(k, 2)

        @pl.when(k + 1 < nmine)
        def _():
            fire_reads(c + _NW, 1 - slot)

        @pl.when(k >= 2)
        def _():
            wait_write(c - 2 * _NW, slot)

        wait_reads(c, slot)
        transpose_into(slot)
        fire_write(c, slot)
        return ()

    lax.fori_loop(0, nmine, loop_body, (), unroll=False)

    @pl.when(nmine >= 2)
    def _():
        wait_write(wid + (nmine - 2) * _NW, lax.rem(nmine - 2, 2))

    @pl.when(nmine >= 1)
    def _():
        wait_write(wid + (nmine - 1) * _NW, lax.rem(nmine - 1, 2))


def _lookup_body(idx_hbm, big_hbm, resid_hbm, out_hbm,
                 idx_v, rows_v, resid_v, gsem):
    wid = lax.axis_index("s") * _NC + lax.axis_index("c")
    base = wid * _BPW
    pltpu.sync_copy(idx_hbm.at[pl.ds(base, _BPW)], idx_v)
    pltpu.sync_copy(resid_hbm, resid_v)
    lane = jnp.arange(_L, dtype=jnp.int32)

    def chunk_gather(i, _):
        idxvec = idx_v[pl.ds(i * _L, _L)]
        pltpu.async_copy(
            big_hbm.at[idxvec], rows_v.at[pl.ds(i * _L, _L), :], gsem
        )
        return ()

    lax.fori_loop(0, _BPW // _L, chunk_gather, (), unroll=False)

    def chunk_drain(i, _):
        idxvec = idx_v[pl.ds(i * _L, _L)]
        pltpu.make_async_copy(
            big_hbm.at[idxvec], rows_v.at[pl.ds(i * _L, _L), :], gsem
        ).wait()
        return ()

    lax.fori_loop(0, _BPW // _L, chunk_drain, (), unroll=False)

    def patch(i, _):
        iv = idx_v[pl.ds(i * _L, _L)]
        m = iv >= jnp.int32(_NCH * _C)
        hit = jnp.max(m.astype(jnp.int32))

        @pl.when(hit > 0)
        def _():
            r = jnp.maximum(iv - jnp.int32(_NCH * _C), 0)
            for c in range(_NT * _D):
                colv = jnp.full((_L,), c, dtype=jnp.int32)
                vals = plsc.load_gather(resid_v, [r, colv], mask=m)
                plsc.store_scatter(rows_v, [i * _L + lane, colv], vals, mask=m)

        return ()

    lax.fori_loop(0, _BPW // _L, patch, (), unroll=False)
    pltpu.sync_copy(rows_v, out_hbm.at[pl.ds(base, _BPW), :])


def kernel(indexes, table0, table1, table2, table3):
    idx = indexes.astype(jnp.int32)
    mesh = plsc.VectorSubcoreMesh(core_axis_name="c", subcore_axis_name="s")
    pack = pl.kernel(
        _pack_body,
        out_type=jax.ShapeDtypeStruct((_V, _NT * _D), jnp.float32),
        mesh=mesh,
        compiler_params=pltpu.CompilerParams(needs_layout_passes=False),
        scratch_types=[
            pltpu.VMEM((_NT, 2, _D, _C), jnp.float32),
            pltpu.VMEM((2, _C, _NT * _D), jnp.float32),
            pltpu.SemaphoreType.DMA,
            pltpu.SemaphoreType.DMA,
        ],
    )
    big = pack(table0.T, table1.T, table2.T, table3.T)
    resid = jnp.concatenate(
        [t[_NCH * _C:, :] for t in (table0, table1, table2, table3)], axis=1
    )

    lookup = pl.kernel(
        _lookup_body,
        out_type=jax.ShapeDtypeStruct((_B, _NT * _D), jnp.float32),
        mesh=mesh,
        compiler_params=pltpu.CompilerParams(needs_layout_passes=False),
        scratch_types=[
            pltpu.VMEM((_BPW,), jnp.int32),
            pltpu.VMEM((_BPW, _NT * _D), jnp.float32),
            pltpu.VMEM((_TAIL, _NT * _D), jnp.float32),
            pltpu.SemaphoreType.DMA,
        ],
    )
    return lookup(idx, big, resid)
